# baseline (speedup 1.0000x reference)
import functools

import jax
import jax.numpy as jnp
from jax import lax
from jax.experimental import pallas as pl
from jax.experimental.pallas import tpu as pltpu
from jax.experimental.pallas import tpu_sc as plsc

_N_NODES = 100000
_NODE_DIM = 128
_HIDDEN_DIM = 64
_NUM_SEGMENTS = 512


_ROWS = 4096
_PHASES = (9, 12, 4)
_N_PAD = _ROWS * sum(_PHASES)
_PK = _NODE_DIM + 3


def _mlp_body(base_block, x_ref, p_ref, out_ref):
    i = base_block + pl.program_id(0)
    x = x_ref[...]
    w1t = p_ref[:, 0:_NODE_DIM]
    b1c = p_ref[:, _NODE_DIM:_NODE_DIM + 1]
    w2c = p_ref[:, _NODE_DIM + 1:_NODE_DIM + 2]
    b2s = p_ref[0, _NODE_DIM + 2]
    ht = lax.dot_general(w1t, x, (((1,), (1,)), ((), ())),
                         preferred_element_type=jnp.float32)
    ht = ht + b1c
    ht = ht * jax.nn.sigmoid(ht)
    e = lax.dot_general(w2c, ht, (((0,), (0,)), ((), ())),
                        preferred_element_type=jnp.float32)
    e = e + b2s
    row = i * _ROWS + lax.broadcasted_iota(jnp.int32, (1, _ROWS), 1)
    e = jnp.where(row < _N_NODES, e, 0.0)
    out_ref[...] = e.reshape(_ROWS)


def _mlp(x, packed, base_block, num_blocks):
    return pl.pallas_call(
        functools.partial(_mlp_body, base_block),
        grid=(num_blocks,),
        in_specs=[
            pl.BlockSpec((_ROWS, _NODE_DIM), lambda i: (i + base_block, 0)),
            pl.BlockSpec((_HIDDEN_DIM, _PK), lambda i: (0, 0)),
        ],
        out_specs=pl.BlockSpec((_ROWS,), lambda i: (i,)),
        out_shape=jax.ShapeDtypeStruct((num_blocks * _ROWS,), jnp.float32),
    )(x, packed)


_NW = 16
_SEG_PER_W = _NUM_SEGMENTS // _NW
_LANES = 16


def _segsum_body(chunk, base_off,
                 e_hbm, b_hbm, prev_hbm, out_hbm,
                 e_v, b_v, accf_v, acc_v, tmp_v, res_v, prev_v,
                 shared, sem_e, sem_b):
    wid = lax.axis_index("s")
    base = wid * chunk
    cp_e = pltpu.async_copy(e_hbm.at[pl.ds(base, chunk)], e_v, sem_e)
    cp_b = pltpu.async_copy(b_hbm.at[pl.ds(base_off + base, chunk)], b_v, sem_b)
    col = wid * _SEG_PER_W
    cp_p = pltpu.async_copy(prev_hbm.at[pl.ds(col, _SEG_PER_W)], prev_v, sem_b)

    zero = jnp.zeros((16,), jnp.float32)
    lane_off = lax.iota(jnp.int32, 16) * _NUM_SEGMENTS

    def zbody(j, carry):
        for u in range(4):
            accf_v[pl.ds(j * 64 + u * 16, 16)] = zero
        return carry

    lax.fori_loop(0, _LANES * _NUM_SEGMENTS // 64, zbody, 0)
    cp_e.wait()
    cp_b.wait()
    cp_p.wait()

    def body(i, carry):
        for u in range(2):
            k = i * 2 + u
            idx = b_v[pl.ds(k * 16, 16)] + lane_off
            v = e_v[pl.ds(k * 16, 16)]
            plsc.addupdate_scatter(accf_v, [idx], v)
        return carry

    lax.fori_loop(0, chunk // 32, body, 0)

    def rbody(j, carry):
        s = zero
        for r in range(_LANES):
            s = s + accf_v[pl.ds(r * _NUM_SEGMENTS + j * 16, 16)]
        acc_v[pl.ds(j * 16, 16)] = s
        return carry

    lax.fori_loop(0, _NUM_SEGMENTS // 16, rbody, 0)

    pltpu.sync_copy(acc_v, shared.at[wid])
    plsc.subcore_barrier()

    cps = [pltpu.async_copy(shared.at[t, pl.ds(col, _SEG_PER_W)], tmp_v.at[t], sem_e)
           for t in range(_NW)]
    for cp in cps:
        cp.wait()
    for q in range(_SEG_PER_W // 16):
        s = prev_v[pl.ds(q * 16, 16)]
        for t in range(_NW):
            s = s + tmp_v[t, pl.ds(q * 16, 16)]
        res_v[pl.ds(q * 16, 16)] = s
    pltpu.sync_copy(res_v, out_hbm.at[pl.ds(col, _SEG_PER_W)])


def _segment_sum(e_phase, b_pad, prev, base_off):
    chunk = e_phase.shape[0] // _NW
    mesh = plsc.VectorSubcoreMesh(
        core_axis_name="c", subcore_axis_name="s", num_cores=1
    )
    return pl.kernel(
        functools.partial(_segsum_body, chunk, base_off),
        mesh=mesh,
        out_type=jax.ShapeDtypeStruct((_NUM_SEGMENTS,), jnp.float32),
        scratch_types=[
            pltpu.VMEM((chunk,), jnp.float32),
            pltpu.VMEM((chunk,), jnp.int32),
            pltpu.VMEM((_LANES * _NUM_SEGMENTS,), jnp.float32),
            pltpu.VMEM((_NUM_SEGMENTS,), jnp.float32),
            pltpu.VMEM((_NW, _SEG_PER_W), jnp.float32),
            pltpu.VMEM((_SEG_PER_W,), jnp.float32),
            pltpu.VMEM((_SEG_PER_W,), jnp.float32),
            pltpu.VMEM_SHARED((_NW, _NUM_SEGMENTS), jnp.float32),
            pltpu.SemaphoreType.DMA,
            pltpu.SemaphoreType.DMA,
        ],
        compiler_params=pltpu.CompilerParams(needs_layout_passes=False),
    )(e_phase, b_pad, prev)


def kernel(node_invariant, batch, W1, b1, W2, b2):
    packed = jnp.concatenate(
        [W1.T, b1.reshape(_HIDDEN_DIM, 1), W2,
         jnp.broadcast_to(b2, (_HIDDEN_DIM, 1))], axis=1)
    b_pad = jnp.pad(batch.astype(jnp.int32), (0, _N_PAD - _N_NODES))
    total = jnp.zeros((_NUM_SEGMENTS,), jnp.float32)
    es = []
    base = 0
    for nb in _PHASES:
        e = _mlp(node_invariant, packed, base, nb)
        total = _segment_sum(e, b_pad, total, base * _ROWS)
        es.append(e)
        base += nb
    atomic = jnp.concatenate(es)[:_N_NODES].reshape(_N_NODES, 1)
    return (total.reshape(_NUM_SEGMENTS, 1), atomic)


# speedup vs baseline: 1.0798x; 1.0798x over previous
import functools

import jax
import jax.numpy as jnp
from jax import lax
from jax.experimental import pallas as pl
from jax.experimental.pallas import tpu as pltpu
from jax.experimental.pallas import tpu_sc as plsc

_N_NODES = 100000
_NODE_DIM = 128
_HIDDEN_DIM = 64
_NUM_SEGMENTS = 512


_ROWS = 4096
_BLOCKS1 = 15
_BLOCKS2 = 10
_N1 = _BLOCKS1 * _ROWS
_N2 = _BLOCKS2 * _ROWS
_N_PAD = _N1 + _N2
_PK = _NODE_DIM + 3


def _mlp_body(base_block, x_ref, p_ref, out_ref):
    i = base_block + pl.program_id(0)
    x = x_ref[...]
    w1t = p_ref[:, 0:_NODE_DIM]
    b1c = p_ref[:, _NODE_DIM:_NODE_DIM + 1]
    w2c = p_ref[:, _NODE_DIM + 1:_NODE_DIM + 2]
    b2s = p_ref[0, _NODE_DIM + 2]
    ht = lax.dot_general(w1t, x, (((1,), (1,)), ((), ())),
                         preferred_element_type=jnp.float32)
    ht = ht + b1c
    ht = ht * jax.nn.sigmoid(ht)
    e = lax.dot_general(w2c, ht, (((0,), (0,)), ((), ())),
                        preferred_element_type=jnp.float32)
    e = e + b2s
    row = i * _ROWS + lax.broadcasted_iota(jnp.int32, (1, _ROWS), 1)
    e = jnp.where(row < _N_NODES, e, 0.0)
    out_ref[...] = e.reshape(_ROWS)


def _mlp(x, packed, base_block, num_blocks):
    return pl.pallas_call(
        functools.partial(_mlp_body, base_block),
        grid=(num_blocks,),
        in_specs=[
            pl.BlockSpec((_ROWS, _NODE_DIM), lambda i: (i + base_block, 0)),
            pl.BlockSpec((_HIDDEN_DIM, _PK), lambda i: (0, 0)),
        ],
        out_specs=pl.BlockSpec((_ROWS,), lambda i: (i,)),
        out_shape=jax.ShapeDtypeStruct((num_blocks * _ROWS,), jnp.float32),
    )(x, packed)


_NW = 16
_SEG_PER_W = _NUM_SEGMENTS // _NW
_LANES = 16


def _segsum_body(chunk, nvec2, safe, safe_nvec2, base_off,
                 e_hbm, b_hbm, prev_hbm, out_hbm,
                 e_v, b_v, accf_v, acc_v, tmp_v, res_v, prev_v,
                 shared, sem_e, sem_b):
    wid = lax.axis_index("s")
    base = wid * chunk
    cp_e = pltpu.async_copy(e_hbm.at[pl.ds(base, chunk)], e_v, sem_e)
    cp_b = pltpu.async_copy(b_hbm.at[pl.ds(base_off + base, safe)],
                            b_v.at[pl.ds(0, safe)], sem_b)

    col = wid * _SEG_PER_W
    cp_p = pltpu.async_copy(prev_hbm.at[pl.ds(col, _SEG_PER_W)], prev_v, sem_b)

    if safe != chunk:
        @pl.when(wid < _NW - 1)
        def _():
            pltpu.async_copy(
                b_hbm.at[pl.ds(base_off + base + safe, chunk - safe)],
                b_v.at[pl.ds(safe, chunk - safe)], sem_b).wait()

    zero = jnp.zeros((16,), jnp.float32)
    lane_off = lax.iota(jnp.int32, 16) * _NUM_SEGMENTS

    def zbody(j, carry):
        for u in range(4):
            accf_v[pl.ds(j * 64 + u * 16, 16)] = zero
        return carry

    lax.fori_loop(0, _LANES * _NUM_SEGMENTS // 64, zbody, 0)
    cp_e.wait()
    cp_b.wait()
    cp_p.wait()

    def body(i, carry):
        for u in range(2):
            k = i * 2 + u
            idx = b_v[pl.ds(k * 16, 16)] + lane_off
            v = e_v[pl.ds(k * 16, 16)]
            plsc.addupdate_scatter(accf_v, [idx], v)
        return carry

    nvec = jnp.where(wid < _NW - 1, nvec2, safe_nvec2)
    lax.fori_loop(0, nvec, body, 0)

    def rbody(j, carry):
        s = zero
        for r in range(_LANES):
            s = s + accf_v[pl.ds(r * _NUM_SEGMENTS + j * 16, 16)]
        acc_v[pl.ds(j * 16, 16)] = s
        return carry

    lax.fori_loop(0, _NUM_SEGMENTS // 16, rbody, 0)

    pltpu.sync_copy(acc_v, shared.at[wid])
    plsc.subcore_barrier()

    cps = [pltpu.async_copy(shared.at[t, pl.ds(col, _SEG_PER_W)], tmp_v.at[t], sem_e)
           for t in range(_NW)]
    for cp in cps:
        cp.wait()
    for q in range(_SEG_PER_W // 16):
        s = prev_v[pl.ds(q * 16, 16)]
        for t in range(_NW):
            s = s + tmp_v[t, pl.ds(q * 16, 16)]
        res_v[pl.ds(q * 16, 16)] = s
    pltpu.sync_copy(res_v, out_hbm.at[pl.ds(col, _SEG_PER_W)])


def _segment_sum(e_pad, b, prev, n, base_off):
    chunk = n // _NW
    n_valid = min(_N_NODES - base_off, n)
    safe = n_valid - (_NW - 1) * chunk
    mesh = plsc.VectorSubcoreMesh(
        core_axis_name="c", subcore_axis_name="s", num_cores=1
    )
    body = functools.partial(_segsum_body, chunk, chunk // 32, safe,
                             safe // 32, base_off)
    return pl.kernel(
        body,
        mesh=mesh,
        out_type=jax.ShapeDtypeStruct((_NUM_SEGMENTS,), jnp.float32),
        scratch_types=[
            pltpu.VMEM((chunk,), jnp.float32),
            pltpu.VMEM((chunk,), jnp.int32),
            pltpu.VMEM((_LANES * _NUM_SEGMENTS,), jnp.float32),
            pltpu.VMEM((_NUM_SEGMENTS,), jnp.float32),
            pltpu.VMEM((_NW, _SEG_PER_W), jnp.float32),
            pltpu.VMEM((_SEG_PER_W,), jnp.float32),
            pltpu.VMEM((_SEG_PER_W,), jnp.float32),
            pltpu.VMEM_SHARED((_NW, _NUM_SEGMENTS), jnp.float32),
            pltpu.SemaphoreType.DMA,
            pltpu.SemaphoreType.DMA,
        ],
        compiler_params=pltpu.CompilerParams(needs_layout_passes=False),
    )(e_pad, b, prev)


def kernel(node_invariant, batch, W1, b1, W2, b2):
    packed = jnp.concatenate(
        [W1.T, b1.reshape(_HIDDEN_DIM, 1), W2,
         jnp.broadcast_to(b2, (_HIDDEN_DIM, 1))], axis=1)
    b32 = batch.astype(jnp.int32)
    e1 = _mlp(node_invariant, packed, 0, _BLOCKS1)
    part1 = _segment_sum(e1, b32, jnp.zeros((_NUM_SEGMENTS,), jnp.float32),
                         _N1, 0)
    e2 = _mlp(node_invariant, packed, _BLOCKS1, _BLOCKS2)
    total = _segment_sum(e2, b32, part1, _N2, _N1)
    atomic = jnp.concatenate([e1, e2[:_N_NODES - _N1]]).reshape(_N_NODES, 1)
    return (total.reshape(_NUM_SEGMENTS, 1), atomic)
